# baseline (speedup 1.0000x reference)
import functools

import jax
import jax.numpy as jnp
from jax import lax
from jax.experimental import pallas as pl
from jax.experimental.pallas import tpu as pltpu
from jax.experimental.pallas import tpu_sc as plsc

M = 36864
K = 256
N = 8192
BM = 4096
BN = 1024
QUANT_W = 4096

NC = 2
NS = 16
NW = NC * NS
ROWS_PER_W = M // NW
CHUNK = 128
NCHUNK = ROWS_PER_W // CHUNK


def _normalize_cb_body(cb_ref, cbn_ref, cbsq_ref):
    cb = cb_ref[...]
    norm = jnp.sqrt(jnp.sum(cb * cb, axis=1, keepdims=True))
    cbn = cb / jnp.maximum(norm, 1e-12)
    cbn_ref[...] = cbn
    cbsq_ref[...] = jnp.sum(cbn * cbn, axis=1, keepdims=True)


def _rtne_bf16(v):
    u = lax.bitcast_convert_type(v, jnp.uint32)
    lsb = (u >> 16) & jnp.uint32(1)
    r = (u + jnp.uint32(0x7FFF) + lsb) & jnp.uint32(0xFFFF0000)
    return lax.bitcast_convert_type(r, jnp.float32)


def _argmin_body(x_ref, cbnt_ref, cbsq_ref, idx_ref, loss_ref, xn_s, xsq_s,
                 rmin_s, rsel_s, rarg_s, lacc_s):
    i = pl.program_id(0)
    j = pl.program_id(1)
    ni = pl.num_programs(0)
    nj = pl.num_programs(1)

    @pl.when(j == 0)
    def _():
        xb = x_ref[...]
        norm = jnp.sqrt(jnp.sum(xb * xb, axis=1, keepdims=True))
        xn = xb / jnp.maximum(norm, 1e-12)
        xn_s[...] = xn.astype(jnp.bfloat16)
        xsq_s[...] = jnp.sum(xn * xn, axis=1, keepdims=True)
        rmin_s[...] = jnp.full((BM, 1), jnp.inf, jnp.float32)
        rsel_s[...] = jnp.full((BM, 1), jnp.inf, jnp.float32)
        rarg_s[...] = jnp.zeros((BM, 1), jnp.int32)

    @pl.when(jnp.logical_and(i == 0, j == 0))
    def _():
        lacc_s[0] = 0.0

    xn = xn_s[...]
    cbnt_b = cbnt_ref[...]
    dot = lax.dot_general(xn, cbnt_b, (((1,), (0,)), ((), ())),
                          preferred_element_type=jnp.float32)
    d = (xsq_s[...] + cbsq_ref[...]) - 2.0 * dot

    lmin = jnp.min(d, axis=1, keepdims=True)
    cols = lax.broadcasted_iota(jnp.int32, (BM, BN), 1)
    larg = jnp.min(jnp.where(d == lmin, cols, BN), axis=1, keepdims=True)
    larg = larg + j * BN
    upd = lmin < rmin_s[...]
    rarg_s[...] = jnp.where(upd, larg, rarg_s[...])
    rmin_s[...] = jnp.where(upd, lmin, rmin_s[...])
    rsel_s[...] = jnp.where(upd, lmin, rsel_s[...])

    @pl.when(j == (QUANT_W // BN) - 1)
    def _():
        rmin_s[...] = _rtne_bf16(rmin_s[...])

    @pl.when(j == nj - 1)
    def _():
        idx_ref[...] = rarg_s[...]
        lacc_s[0] += jnp.sum(rsel_s[...])

    @pl.when(jnp.logical_and(i == ni - 1, j == nj - 1))
    def _():
        loss_ref[0] = lacc_s[0] * (1.25 / float(M * K))


def _gather_sc_body(cbn_hbm, idx_hbm, out_hbm, idx_v, rows0, rows1, sem0,
                    sem1):
    wid = lax.axis_index("s") * NC + lax.axis_index("c")
    pltpu.sync_copy(idx_hbm.at[pl.ds(wid * ROWS_PER_W, ROWS_PER_W)], idx_v)
    bufs = (rows0, rows1)
    sems = (sem0, sem1)
    copies = [pltpu.async_copy(
        cbn_hbm.at[idx_v.at[pl.ds(0, CHUNK)]], bufs[0], sems[0])]
    for jj in range(NCHUNK):
        if jj + 1 < NCHUNK:
            copies.append(pltpu.async_copy(
                cbn_hbm.at[idx_v.at[pl.ds((jj + 1) * CHUNK, CHUNK)]],
                bufs[(jj + 1) % 2], sems[(jj + 1) % 2]))
        copies[jj].wait()
        pltpu.sync_copy(bufs[jj % 2],
                        out_hbm.at[pl.ds(wid * ROWS_PER_W + jj * CHUNK, CHUNK)])


@functools.cache
def _gather_sc():
    mesh = plsc.VectorSubcoreMesh(
        core_axis_name="c", subcore_axis_name="s", num_cores=NC,
        num_subcores=NS)
    return pl.kernel(
        _gather_sc_body,
        out_type=jax.ShapeDtypeStruct((M, K), jnp.float32),
        mesh=mesh,
        scratch_types=[
            pltpu.VMEM((ROWS_PER_W,), jnp.int32),
            pltpu.VMEM((CHUNK, K), jnp.float32),
            pltpu.VMEM((CHUNK, K), jnp.float32),
            pltpu.SemaphoreType.DMA,
            pltpu.SemaphoreType.DMA,
        ],
    )


def _normalize_cb(codebook):
    return pl.pallas_call(
        _normalize_cb_body,
        grid=(N // BN,),
        in_specs=[pl.BlockSpec((BN, K), lambda j: (j, 0))],
        out_specs=[
            pl.BlockSpec((BN, K), lambda j: (j, 0)),
            pl.BlockSpec((BN, 1), lambda j: (j, 0)),
        ],
        out_shape=[
            jax.ShapeDtypeStruct((N, K), jnp.float32),
            jax.ShapeDtypeStruct((N, 1), jnp.float32),
        ],
    )(codebook)


def _argmin_call(x, cbnt, cbsq_row):
    return pl.pallas_call(
        _argmin_body,
        grid=(M // BM, N // BN),
        in_specs=[
            pl.BlockSpec((BM, K), lambda i, j: (i, 0)),
            pl.BlockSpec((K, BN), lambda i, j: (0, j)),
            pl.BlockSpec((1, BN), lambda i, j: (0, j)),
        ],
        out_specs=[
            pl.BlockSpec((BM, 1), lambda i, j: (i, 0)),
            pl.BlockSpec(memory_space=pltpu.SMEM),
        ],
        out_shape=[
            jax.ShapeDtypeStruct((M, 1), jnp.int32),
            jax.ShapeDtypeStruct((1,), jnp.float32),
        ],
        scratch_shapes=[
            pltpu.VMEM((BM, K), jnp.bfloat16),
            pltpu.VMEM((BM, 1), jnp.float32),
            pltpu.VMEM((BM, 1), jnp.float32),
            pltpu.VMEM((BM, 1), jnp.float32),
            pltpu.VMEM((BM, 1), jnp.int32),
            pltpu.SMEM((1,), jnp.float32),
        ],
    )(x, cbnt, cbsq_row)


def kernel(x, codebook):
    cbn, cbsq = _normalize_cb(codebook)
    idx2d, loss_raw = _argmin_call(x, cbn.T.astype(jnp.bfloat16), cbsq.T)
    idx = idx2d.reshape((M,))
    quantized = _gather_sc()(cbn, idx)
    loss = loss_raw.reshape(())
    return (quantized, loss, idx)


# speedup vs baseline: 1.0075x; 1.0075x over previous
import functools

import jax
import jax.numpy as jnp
from jax import lax
from jax.experimental import pallas as pl
from jax.experimental.pallas import tpu as pltpu
from jax.experimental.pallas import tpu_sc as plsc

M = 36864
K = 256
N = 8192
BM = 4096
BN = 1024
QUANT_W = 4096

NC = 2
NS = 16
NW = NC * NS
ROWS_PER_W = M // NW
CHUNK = 128
NCHUNK = ROWS_PER_W // CHUNK


def _normalize_cb_body(cb_ref, cbn_ref, cbsq_ref):
    cb = cb_ref[...]
    norm = jnp.sqrt(jnp.sum(cb * cb, axis=1, keepdims=True))
    cbn = cb / jnp.maximum(norm, 1e-12)
    cbn_ref[...] = cbn
    cbsq_ref[...] = jnp.sum(cbn * cbn, axis=1, keepdims=True)


def _rtne_bf16(v):
    u = lax.bitcast_convert_type(v, jnp.uint32)
    lsb = (u >> 16) & jnp.uint32(1)
    r = (u + jnp.uint32(0x7FFF) + lsb) & jnp.uint32(0xFFFF0000)
    return lax.bitcast_convert_type(r, jnp.float32)


def _argmin_body(x_ref, cbnt_ref, cbsq_ref, idx_ref, loss_ref, xn_s, xsq_s,
                 rmin_s, rsel_s, rarg_s, lacc_s):
    i = pl.program_id(0)
    j = pl.program_id(1)
    ni = pl.num_programs(0)
    nj = pl.num_programs(1)

    @pl.when(j == 0)
    def _():
        xb = x_ref[...]
        norm = jnp.sqrt(jnp.sum(xb * xb, axis=1, keepdims=True))
        xn = xb / jnp.maximum(norm, 1e-12)
        xn_s[...] = xn
        xsq_s[...] = jnp.sum(xn * xn, axis=1, keepdims=True)
        rmin_s[...] = jnp.full((BM, 1), jnp.inf, jnp.float32)
        rsel_s[...] = jnp.full((BM, 1), jnp.inf, jnp.float32)
        rarg_s[...] = jnp.zeros((BM, 1), jnp.int32)

    @pl.when(jnp.logical_and(i == 0, j == 0))
    def _():
        lacc_s[0] = 0.0

    xn = xn_s[...]
    cbnt_b = cbnt_ref[...]
    dot = lax.dot_general(xn.astype(jnp.bfloat16), cbnt_b.astype(jnp.bfloat16),
                          (((1,), (0,)), ((), ())),
                          preferred_element_type=jnp.float32)
    d = (xsq_s[...] + cbsq_ref[...]) - 2.0 * dot

    lmin = jnp.min(d, axis=1, keepdims=True)
    cols = lax.broadcasted_iota(jnp.int32, (BM, BN), 1)
    larg = jnp.min(jnp.where(d == lmin, cols, BN), axis=1, keepdims=True)
    larg = larg + j * BN
    upd = lmin < rmin_s[...]
    rarg_s[...] = jnp.where(upd, larg, rarg_s[...])
    rmin_s[...] = jnp.where(upd, lmin, rmin_s[...])
    rsel_s[...] = jnp.where(upd, lmin, rsel_s[...])

    @pl.when(j == (QUANT_W // BN) - 1)
    def _():
        rmin_s[...] = _rtne_bf16(rmin_s[...])

    @pl.when(j == nj - 1)
    def _():
        idx_ref[...] = rarg_s[...]
        lacc_s[0] += jnp.sum(rsel_s[...])

    @pl.when(jnp.logical_and(i == ni - 1, j == nj - 1))
    def _():
        loss_ref[0] = lacc_s[0] * (1.25 / float(M * K))


def _gather_sc_body(cbn_hbm, idx_hbm, out_hbm, idx_v, rows0, rows1, sem0,
                    sem1):
    wid = lax.axis_index("s") * NC + lax.axis_index("c")
    pltpu.sync_copy(idx_hbm.at[pl.ds(wid * ROWS_PER_W, ROWS_PER_W)], idx_v)
    bufs = (rows0, rows1)
    sems = (sem0, sem1)
    copies = [pltpu.async_copy(
        cbn_hbm.at[idx_v.at[pl.ds(0, CHUNK)]], bufs[0], sems[0])]
    for jj in range(NCHUNK):
        if jj + 1 < NCHUNK:
            copies.append(pltpu.async_copy(
                cbn_hbm.at[idx_v.at[pl.ds((jj + 1) * CHUNK, CHUNK)]],
                bufs[(jj + 1) % 2], sems[(jj + 1) % 2]))
        copies[jj].wait()
        pltpu.sync_copy(bufs[jj % 2],
                        out_hbm.at[pl.ds(wid * ROWS_PER_W + jj * CHUNK, CHUNK)])


@functools.cache
def _gather_sc():
    mesh = plsc.VectorSubcoreMesh(
        core_axis_name="c", subcore_axis_name="s", num_cores=NC,
        num_subcores=NS)
    return pl.kernel(
        _gather_sc_body,
        out_type=jax.ShapeDtypeStruct((M, K), jnp.float32),
        mesh=mesh,
        scratch_types=[
            pltpu.VMEM((ROWS_PER_W,), jnp.int32),
            pltpu.VMEM((CHUNK, K), jnp.float32),
            pltpu.VMEM((CHUNK, K), jnp.float32),
            pltpu.SemaphoreType.DMA,
            pltpu.SemaphoreType.DMA,
        ],
    )


def _normalize_cb(codebook):
    return pl.pallas_call(
        _normalize_cb_body,
        grid=(N // BN,),
        in_specs=[pl.BlockSpec((BN, K), lambda j: (j, 0))],
        out_specs=[
            pl.BlockSpec((BN, K), lambda j: (j, 0)),
            pl.BlockSpec((BN, 1), lambda j: (j, 0)),
        ],
        out_shape=[
            jax.ShapeDtypeStruct((N, K), jnp.float32),
            jax.ShapeDtypeStruct((N, 1), jnp.float32),
        ],
    )(codebook)


def _argmin_call(x, cbnt, cbsq_row):
    return pl.pallas_call(
        _argmin_body,
        grid=(M // BM, N // BN),
        in_specs=[
            pl.BlockSpec((BM, K), lambda i, j: (i, 0)),
            pl.BlockSpec((K, BN), lambda i, j: (0, j)),
            pl.BlockSpec((1, BN), lambda i, j: (0, j)),
        ],
        out_specs=[
            pl.BlockSpec((BM, 1), lambda i, j: (i, 0)),
            pl.BlockSpec(memory_space=pltpu.SMEM),
        ],
        out_shape=[
            jax.ShapeDtypeStruct((M, 1), jnp.int32),
            jax.ShapeDtypeStruct((1,), jnp.float32),
        ],
        scratch_shapes=[
            pltpu.VMEM((BM, K), jnp.float32),
            pltpu.VMEM((BM, 1), jnp.float32),
            pltpu.VMEM((BM, 1), jnp.float32),
            pltpu.VMEM((BM, 1), jnp.float32),
            pltpu.VMEM((BM, 1), jnp.int32),
            pltpu.SMEM((1,), jnp.float32),
        ],
    )(x, cbnt, cbsq_row)


def kernel(x, codebook):
    cbn, cbsq = _normalize_cb(codebook)
    idx2d, loss_raw = _argmin_call(x, cbn.T, cbsq.T)
    idx = idx2d.reshape((M,))
    quantized = _gather_sc()(cbn, idx)
    loss = loss_raw.reshape(())
    return (quantized, loss, idx)
